# baseline (speedup 1.0000x reference)
import functools
import jax
import jax.numpy as jnp
from jax import lax
from jax.experimental import pallas as pl
from jax.experimental.pallas import tpu as pltpu
from jax.experimental.pallas import tpu_sc as plsc

DIM = 1024
NUM_EXPERTS = 8
HIDDEN = DIM * 2
TOKENS = 4096
SLOTS = 2 * TOKENS

BT = 128
NB = SLOTS // BT + NUM_EXPERTS
NPAD = NB * BT
NBPAD = 128

NWORKERS = 32
GCHUNK = 64

GR = 64
GC = 128


def _gate_body(x_ref, gw_ref, gb_ref, e1_ref, e2_ref, p1_ref, p2_ref):
    logits = lax.dot_general(
        gw_ref[...], x_ref[...], (((0,), (1,)), ((), ())),
        precision=lax.Precision.DEFAULT,
        preferred_element_type=jnp.float32) + gb_ref[...]
    i1 = jnp.argmax(logits, axis=0, keepdims=True)
    m1 = jnp.max(logits, axis=0, keepdims=True)
    eidx = lax.broadcasted_iota(jnp.int32, logits.shape, 0)
    masked = jnp.where(eidx == i1, -jnp.inf, logits)
    i2 = jnp.argmax(masked, axis=0, keepdims=True)
    m2 = jnp.max(masked, axis=0, keepdims=True)
    p1 = 1.0 / (1.0 + jnp.exp(m2 - m1))
    e1_ref[0] = i1.astype(jnp.int32)
    e2_ref[0] = i2.astype(jnp.int32)
    p1_ref[0] = p1
    p2_ref[0] = 1.0 - p1


def _gate(x, gate_w, gate_b):
    bt = 1024
    nt = TOKENS // bt
    return pl.pallas_call(
        _gate_body,
        grid=(nt,),
        in_specs=[
            pl.BlockSpec((bt, DIM), lambda i: (i, 0)),
            pl.BlockSpec((DIM, NUM_EXPERTS), lambda i: (0, 0)),
            pl.BlockSpec((NUM_EXPERTS, 1), lambda i: (0, 0)),
        ],
        out_specs=[
            pl.BlockSpec((1, 1, bt), lambda i: (i, 0, 0)),
            pl.BlockSpec((1, 1, bt), lambda i: (i, 0, 0)),
            pl.BlockSpec((1, 1, bt), lambda i: (i, 0, 0)),
            pl.BlockSpec((1, 1, bt), lambda i: (i, 0, 0)),
        ],
        out_shape=[
            jax.ShapeDtypeStruct((nt, 1, bt), jnp.int32),
            jax.ShapeDtypeStruct((nt, 1, bt), jnp.int32),
            jax.ShapeDtypeStruct((nt, 1, bt), jnp.float32),
            jax.ShapeDtypeStruct((nt, 1, bt), jnp.float32),
        ],
    )(x, gate_w, gate_b.reshape(NUM_EXPERTS, 1))


def _routing_body(e1_ref, e2_ref, dest_ref, be_ref):
    e1 = jnp.reshape(e1_ref[...], (GR // 2, GC))
    e2 = jnp.reshape(e2_ref[...], (GR // 2, GC))
    ea = jnp.concatenate([e1, e2], axis=0)

    ci = lax.broadcasted_iota(jnp.int32, (GC, GC), 0)
    cj = lax.broadcasted_iota(jnp.int32, (GC, GC), 1)
    tri_inc = (ci <= cj).astype(jnp.bfloat16)
    ri = lax.broadcasted_iota(jnp.int32, (GR, GR), 0)
    rj = lax.broadcasted_iota(jnp.int32, (GR, GR), 1)
    tri_strict = (ri > rj).astype(jnp.bfloat16)

    dest = jnp.zeros((GR, GC), jnp.float32)
    running = jnp.zeros((), jnp.float32)
    cumblk = []
    for e in range(NUM_EXPERTS):
        m = (ea == e).astype(jnp.float32)
        rowcs = lax.dot_general(m.astype(jnp.bfloat16), tri_inc,
                                (((1,), (0,)), ((), ())),
                                preferred_element_type=jnp.float32)
        rowsum = rowcs[:, GC - 1:GC]
        prevrows = lax.dot_general(tri_strict,
                                   rowsum.astype(jnp.bfloat16),
                                   (((1,), (0,)), ((), ())),
                                   preferred_element_type=jnp.float32)
        rank = rowcs - m + prevrows
        cnt = jnp.sum(m)
        dest = dest + m * (running + rank)
        running = running + jnp.ceil(cnt / BT) * BT
        cumblk.append(running / BT)

    dest_ref[...] = dest.astype(jnp.int32)

    bi = lax.broadcasted_iota(jnp.int32, (1, NBPAD), 1).astype(jnp.float32)
    be = jnp.zeros((1, NBPAD), jnp.int32)
    for e in range(NUM_EXPERTS - 1):
        be = be + (bi >= cumblk[e]).astype(jnp.int32)
    used = cumblk[NUM_EXPERTS - 1].astype(jnp.int32)
    be = jnp.where(lax.broadcasted_iota(jnp.int32, (1, NBPAD), 1)
                   == NBPAD - 1, used, be)
    be_ref[...] = be


def _routing(e1, e2):
    nt = TOKENS // 1024
    specs = [pl.BlockSpec((nt, 1, 1024), lambda: (0, 0, 0))] * 2
    return pl.pallas_call(
        _routing_body,
        in_specs=specs,
        out_specs=[
            pl.BlockSpec((GR, GC), lambda: (0, 0)),
            pl.BlockSpec((1, NBPAD), lambda: (0, 0)),
        ],
        out_shape=[
            jax.ShapeDtypeStruct((GR, GC), jnp.int32),
            jax.ShapeDtypeStruct((1, NBPAD), jnp.int32),
        ],
    )(e1, e2)


GD = 32
SLOTS_PER_W = SLOTS // NWORKERS
NCH = SLOTS_PER_W // GD


@functools.lru_cache(maxsize=None)
def _make_sc_dispatch():
    mesh = plsc.VectorSubcoreMesh(core_axis_name="c", subcore_axis_name="s")

    @functools.partial(
        pl.kernel, mesh=mesh,
        out_type=jax.ShapeDtypeStruct((NPAD, DIM), jnp.float32),
        scratch_types=[
            pltpu.VMEM((NCH, GD), jnp.int32),
            pltpu.VMEM((2, GD), jnp.int32),
            pltpu.VMEM((2, GD, DIM), jnp.float32),
            pltpu.SemaphoreType.DMA,
            pltpu.SemaphoreType.DMA,
        ],
    )
    def dispatch_k(x_hbm, dest_hbm, xg_hbm,
                   dest_v, tok_v, rows_v, semg, sems):
        wid = lax.axis_index("s") * 2 + lax.axis_index("c")
        base = wid * SLOTS_PER_W
        pltpu.sync_copy(dest_hbm.at[wid], dest_v)
        pend = []
        for c in range(NCH):
            b = c % 2
            for g in range(GD // 16):
                s16 = (base + c * GD + g * 16 +
                       lax.broadcasted_iota(jnp.int32, (16,), 0))
                tok_v[b, pl.ds(g * 16, 16)] = jnp.where(
                    s16 >= TOKENS, s16 - TOKENS, s16)
            if c >= 2:
                pend[c - 2].wait()
            pltpu.async_copy(x_hbm.at[tok_v.at[b]], rows_v.at[b], semg).wait()
            pend.append(
                pltpu.async_copy(rows_v.at[b], xg_hbm.at[dest_v.at[c]], sems))
        for h in pend[-2:]:
            h.wait()

    return dispatch_k


def _sc_dispatch(x, dest3):
    return _make_sc_dispatch()(x, dest3)


@functools.lru_cache(maxsize=None)
def _make_sc_gather(n_rows, dim, dtype):
    rows_per_w = n_rows // NWORKERS
    n_chunks = rows_per_w // GD
    mesh = plsc.VectorSubcoreMesh(core_axis_name="c", subcore_axis_name="s")

    @functools.partial(
        pl.kernel, mesh=mesh,
        out_type=jax.ShapeDtypeStruct((n_rows, dim), dtype),
        scratch_types=[
            pltpu.VMEM((n_chunks, GD), jnp.int32),
            pltpu.VMEM((2, GD, dim), dtype),
            pltpu.SemaphoreType.DMA,
            pltpu.SemaphoreType.DMA,
        ],
    )
    def gather_k(table_hbm, idx_hbm, out_hbm, idx_v, rows_v, semg, semo):
        wid = lax.axis_index("s") * 2 + lax.axis_index("c")
        base = wid * rows_per_w
        pltpu.sync_copy(idx_hbm.at[wid], idx_v)
        pend = []
        for c in range(n_chunks):
            b = c % 2
            if c >= 2:
                pend[c - 2].wait()
            pltpu.async_copy(table_hbm.at[idx_v.at[c]], rows_v.at[b],
                             semg).wait()
            pend.append(
                pltpu.async_copy(rows_v.at[b],
                                 out_hbm.at[pl.ds(base + c * GD, GD)], semo))
        for h in pend[-2:]:
            h.wait()

    return gather_k


def _sc_gather_y(table, idx):
    return _make_sc_gather(SLOTS, DIM, jnp.float32)(table, idx)


def _ffn_body(be_ref, xg_ref, w1_ref, b1_ref, w2_ref, b2_ref, y_ref):
    @pl.when(pl.program_id(0) < be_ref[NBPAD - 1])
    def _compute():
        xb = xg_ref[...].astype(jnp.bfloat16)
        h = lax.dot_general(
            xb, w1_ref[0], (((1,), (0,)), ((), ())),
            preferred_element_type=jnp.float32) + b1_ref[0]
        h = 0.5 * h * (1.0 + lax.erf(h * 0.7071067811865476))
        y = lax.dot_general(
            h.astype(jnp.bfloat16), w2_ref[0], (((1,), (0,)), ((), ())),
            preferred_element_type=jnp.float32) + b2_ref[0]
        y_ref[...] = y


def _ffn(block_expert, xg, w1, b1, w2, b2):
    grid_spec = pltpu.PrefetchScalarGridSpec(
        num_scalar_prefetch=1,
        grid=(NB,),
        in_specs=[
            pl.BlockSpec((BT, DIM), lambda b, be: (b, 0)),
            pl.BlockSpec((1, DIM, HIDDEN), lambda b, be: (be[b], 0, 0)),
            pl.BlockSpec((1, 1, HIDDEN), lambda b, be: (be[b], 0, 0)),
            pl.BlockSpec((1, HIDDEN, DIM), lambda b, be: (be[b], 0, 0)),
            pl.BlockSpec((1, 1, DIM), lambda b, be: (be[b], 0, 0)),
        ],
        out_specs=pl.BlockSpec((BT, DIM), lambda b, be: (b, 0)),
    )
    return pl.pallas_call(
        _ffn_body,
        grid_spec=grid_spec,
        out_shape=jax.ShapeDtypeStruct((NPAD, DIM), jnp.float32),
    )(block_expert, xg,
      w1.astype(jnp.bfloat16), b1.reshape(NUM_EXPERTS, 1, HIDDEN),
      w2.astype(jnp.bfloat16), b2.reshape(NUM_EXPERTS, 1, DIM))


def _add_body(a_ref, b_ref, pa_ref, pb_ref, o_ref):
    o_ref[...] = a_ref[...] * pa_ref[...] + b_ref[...] * pb_ref[...]


def _combine_add(y01, p1, p2):
    bt = 512
    return pl.pallas_call(
        _add_body,
        grid=(TOKENS // bt,),
        in_specs=[
            pl.BlockSpec((bt, DIM), lambda i: (i, 0)),
            pl.BlockSpec((bt, DIM), lambda i: (i + TOKENS // bt, 0)),
            pl.BlockSpec((bt, 1), lambda i: (i, 0)),
            pl.BlockSpec((bt, 1), lambda i: (i, 0)),
        ],
        out_specs=pl.BlockSpec((bt, DIM), lambda i: (i, 0)),
        out_shape=jax.ShapeDtypeStruct((TOKENS, DIM), jnp.float32),
    )(y01, y01, p1.reshape(TOKENS, 1), p2.reshape(TOKENS, 1))


@jax.jit
def _moe(x, gate_w, gate_b, w1, b1, w2, b2):
    e1, e2, p1, p2 = _gate(x, gate_w, gate_b)
    dest2d, bexp = _routing(e1, e2)
    dest3 = dest2d.reshape(NWORKERS, NCH, GD)
    xg = _sc_dispatch(x, dest3)
    y = _ffn(bexp.reshape(NBPAD), xg, w1, b1, w2, b2)
    y01 = _sc_gather_y(y, dest3)
    return _combine_add(y01, p1, p2)


def kernel(x, gate_w, gate_b, w1, b1, w2, b2):
    return _moe(x, gate_w, gate_b, w1, b1, w2, b2)


# speedup vs baseline: 1.0522x; 1.0522x over previous
import functools
import jax
import jax.numpy as jnp
from jax import lax
from jax.experimental import pallas as pl
from jax.experimental.pallas import tpu as pltpu
from jax.experimental.pallas import tpu_sc as plsc

DIM = 1024
NUM_EXPERTS = 8
HIDDEN = DIM * 2
TOKENS = 4096
SLOTS = 2 * TOKENS

BT = 256
NB = SLOTS // BT + NUM_EXPERTS
NPAD = NB * BT
NBPAD = 128

NWORKERS = 32
GCHUNK = 64

GR = 64
GC = 128


def _gate_body(x_ref, gw_ref, gb_ref, e1_ref, e2_ref, p1_ref, p2_ref):
    logits = lax.dot_general(
        gw_ref[...], x_ref[...], (((0,), (1,)), ((), ())),
        precision=lax.Precision.DEFAULT,
        preferred_element_type=jnp.float32) + gb_ref[...]
    i1 = jnp.argmax(logits, axis=0, keepdims=True)
    m1 = jnp.max(logits, axis=0, keepdims=True)
    eidx = lax.broadcasted_iota(jnp.int32, logits.shape, 0)
    masked = jnp.where(eidx == i1, -jnp.inf, logits)
    i2 = jnp.argmax(masked, axis=0, keepdims=True)
    m2 = jnp.max(masked, axis=0, keepdims=True)
    p1 = 1.0 / (1.0 + jnp.exp(m2 - m1))
    e1_ref[0] = i1.astype(jnp.int32)
    e2_ref[0] = i2.astype(jnp.int32)
    p1_ref[0] = p1
    p2_ref[0] = 1.0 - p1


def _gate(x, gate_w, gate_b):
    bt = 1024
    nt = TOKENS // bt
    return pl.pallas_call(
        _gate_body,
        grid=(nt,),
        in_specs=[
            pl.BlockSpec((bt, DIM), lambda i: (i, 0)),
            pl.BlockSpec((DIM, NUM_EXPERTS), lambda i: (0, 0)),
            pl.BlockSpec((NUM_EXPERTS, 1), lambda i: (0, 0)),
        ],
        out_specs=[
            pl.BlockSpec((1, 1, bt), lambda i: (i, 0, 0)),
            pl.BlockSpec((1, 1, bt), lambda i: (i, 0, 0)),
            pl.BlockSpec((1, 1, bt), lambda i: (i, 0, 0)),
            pl.BlockSpec((1, 1, bt), lambda i: (i, 0, 0)),
        ],
        out_shape=[
            jax.ShapeDtypeStruct((nt, 1, bt), jnp.int32),
            jax.ShapeDtypeStruct((nt, 1, bt), jnp.int32),
            jax.ShapeDtypeStruct((nt, 1, bt), jnp.float32),
            jax.ShapeDtypeStruct((nt, 1, bt), jnp.float32),
        ],
    )(x, gate_w, gate_b.reshape(NUM_EXPERTS, 1))


def _routing_body(e1_ref, e2_ref, dest_ref, be_ref):
    e1 = jnp.reshape(e1_ref[...], (GR // 2, GC))
    e2 = jnp.reshape(e2_ref[...], (GR // 2, GC))
    ea = jnp.concatenate([e1, e2], axis=0)

    ci = lax.broadcasted_iota(jnp.int32, (GC, GC), 0)
    cj = lax.broadcasted_iota(jnp.int32, (GC, GC), 1)
    tri_inc = (ci <= cj).astype(jnp.bfloat16)
    ri = lax.broadcasted_iota(jnp.int32, (GR, GR), 0)
    rj = lax.broadcasted_iota(jnp.int32, (GR, GR), 1)
    tri_strict = (ri > rj).astype(jnp.bfloat16)

    dest = jnp.zeros((GR, GC), jnp.float32)
    running = jnp.zeros((), jnp.float32)
    cumblk = []
    for e in range(NUM_EXPERTS):
        m = (ea == e).astype(jnp.float32)
        rowcs = lax.dot_general(m.astype(jnp.bfloat16), tri_inc,
                                (((1,), (0,)), ((), ())),
                                preferred_element_type=jnp.float32)
        rowsum = rowcs[:, GC - 1:GC]
        prevrows = lax.dot_general(tri_strict,
                                   rowsum.astype(jnp.bfloat16),
                                   (((1,), (0,)), ((), ())),
                                   preferred_element_type=jnp.float32)
        rank = rowcs - m + prevrows
        cnt = jnp.sum(m)
        dest = dest + m * (running + rank)
        running = running + jnp.ceil(cnt / BT) * BT
        cumblk.append(running / BT)

    dest_ref[...] = dest.astype(jnp.int32)

    bi = lax.broadcasted_iota(jnp.int32, (1, NBPAD), 1).astype(jnp.float32)
    be = jnp.zeros((1, NBPAD), jnp.int32)
    for e in range(NUM_EXPERTS - 1):
        be = be + (bi >= cumblk[e]).astype(jnp.int32)
    used = cumblk[NUM_EXPERTS - 1].astype(jnp.int32)
    be = jnp.where(lax.broadcasted_iota(jnp.int32, (1, NBPAD), 1)
                   == NBPAD - 1, used, be)
    be_ref[...] = be


def _routing(e1, e2):
    nt = TOKENS // 1024
    specs = [pl.BlockSpec((nt, 1, 1024), lambda: (0, 0, 0))] * 2
    return pl.pallas_call(
        _routing_body,
        in_specs=specs,
        out_specs=[
            pl.BlockSpec((GR, GC), lambda: (0, 0)),
            pl.BlockSpec((1, NBPAD), lambda: (0, 0)),
        ],
        out_shape=[
            jax.ShapeDtypeStruct((GR, GC), jnp.int32),
            jax.ShapeDtypeStruct((1, NBPAD), jnp.int32),
        ],
    )(e1, e2)


GD = 32
SLOTS_PER_W = SLOTS // NWORKERS
NCH = SLOTS_PER_W // GD


@functools.lru_cache(maxsize=None)
def _make_sc_dispatch():
    mesh = plsc.VectorSubcoreMesh(core_axis_name="c", subcore_axis_name="s")

    @functools.partial(
        pl.kernel, mesh=mesh,
        out_type=jax.ShapeDtypeStruct((NPAD, DIM), jnp.float32),
        scratch_types=[
            pltpu.VMEM((NCH, GD), jnp.int32),
            pltpu.VMEM((2, GD), jnp.int32),
            pltpu.VMEM((2, GD, DIM), jnp.float32),
            pltpu.SemaphoreType.DMA,
            pltpu.SemaphoreType.DMA,
        ],
    )
    def dispatch_k(x_hbm, dest_hbm, xg_hbm,
                   dest_v, tok_v, rows_v, semg, sems):
        wid = lax.axis_index("s") * 2 + lax.axis_index("c")
        base = wid * SLOTS_PER_W
        pltpu.sync_copy(dest_hbm.at[wid], dest_v)
        pend = []
        for c in range(NCH):
            b = c % 2
            for g in range(GD // 16):
                s16 = (base + c * GD + g * 16 +
                       lax.broadcasted_iota(jnp.int32, (16,), 0))
                tok_v[b, pl.ds(g * 16, 16)] = jnp.where(
                    s16 >= TOKENS, s16 - TOKENS, s16)
            if c >= 2:
                pend[c - 2].wait()
            pltpu.async_copy(x_hbm.at[tok_v.at[b]], rows_v.at[b], semg).wait()
            pend.append(
                pltpu.async_copy(rows_v.at[b], xg_hbm.at[dest_v.at[c]], sems))
        for h in pend[-2:]:
            h.wait()

    return dispatch_k


def _sc_dispatch(x, dest3):
    return _make_sc_dispatch()(x, dest3)


@functools.lru_cache(maxsize=None)
def _make_sc_gather(n_rows, dim, dtype):
    rows_per_w = n_rows // NWORKERS
    n_chunks = rows_per_w // GD
    mesh = plsc.VectorSubcoreMesh(core_axis_name="c", subcore_axis_name="s")

    @functools.partial(
        pl.kernel, mesh=mesh,
        out_type=jax.ShapeDtypeStruct((n_rows, dim), dtype),
        scratch_types=[
            pltpu.VMEM((n_chunks, GD), jnp.int32),
            pltpu.VMEM((2, GD, dim), dtype),
            pltpu.SemaphoreType.DMA,
            pltpu.SemaphoreType.DMA,
        ],
    )
    def gather_k(table_hbm, idx_hbm, out_hbm, idx_v, rows_v, semg, semo):
        wid = lax.axis_index("s") * 2 + lax.axis_index("c")
        base = wid * rows_per_w
        pltpu.sync_copy(idx_hbm.at[wid], idx_v)
        pend = []
        for c in range(n_chunks):
            b = c % 2
            if c >= 2:
                pend[c - 2].wait()
            pltpu.async_copy(table_hbm.at[idx_v.at[c]], rows_v.at[b],
                             semg).wait()
            pend.append(
                pltpu.async_copy(rows_v.at[b],
                                 out_hbm.at[pl.ds(base + c * GD, GD)], semo))
        for h in pend[-2:]:
            h.wait()

    return gather_k


def _sc_gather_y(table, idx):
    return _make_sc_gather(SLOTS, DIM, jnp.float32)(table, idx)


def _ffn_body(be_ref, xg_ref, w1_ref, b1_ref, w2_ref, b2_ref, y_ref):
    @pl.when(pl.program_id(0) < be_ref[NBPAD - 1])
    def _compute():
        xb = xg_ref[...].astype(jnp.bfloat16)
        h = lax.dot_general(
            xb, w1_ref[0], (((1,), (0,)), ((), ())),
            preferred_element_type=jnp.float32) + b1_ref[0]
        h = 0.5 * h * (1.0 + lax.erf(h * 0.7071067811865476))
        y = lax.dot_general(
            h.astype(jnp.bfloat16), w2_ref[0], (((1,), (0,)), ((), ())),
            preferred_element_type=jnp.float32) + b2_ref[0]
        y_ref[...] = y


def _ffn(block_expert, xg, w1, b1, w2, b2):
    grid_spec = pltpu.PrefetchScalarGridSpec(
        num_scalar_prefetch=1,
        grid=(NB,),
        in_specs=[
            pl.BlockSpec((BT, DIM), lambda b, be: (b, 0)),
            pl.BlockSpec((1, DIM, HIDDEN), lambda b, be: (be[b], 0, 0)),
            pl.BlockSpec((1, 1, HIDDEN), lambda b, be: (be[b], 0, 0)),
            pl.BlockSpec((1, HIDDEN, DIM), lambda b, be: (be[b], 0, 0)),
            pl.BlockSpec((1, 1, DIM), lambda b, be: (be[b], 0, 0)),
        ],
        out_specs=pl.BlockSpec((BT, DIM), lambda b, be: (b, 0)),
    )
    return pl.pallas_call(
        _ffn_body,
        grid_spec=grid_spec,
        out_shape=jax.ShapeDtypeStruct((NPAD, DIM), jnp.float32),
    )(block_expert, xg,
      w1.astype(jnp.bfloat16), b1.reshape(NUM_EXPERTS, 1, HIDDEN),
      w2.astype(jnp.bfloat16), b2.reshape(NUM_EXPERTS, 1, DIM))


def _add_body(a_ref, b_ref, pa_ref, pb_ref, o_ref):
    o_ref[...] = a_ref[...] * pa_ref[...] + b_ref[...] * pb_ref[...]


def _combine_add(y01, p1, p2):
    bt = 512
    return pl.pallas_call(
        _add_body,
        grid=(TOKENS // bt,),
        in_specs=[
            pl.BlockSpec((bt, DIM), lambda i: (i, 0)),
            pl.BlockSpec((bt, DIM), lambda i: (i + TOKENS // bt, 0)),
            pl.BlockSpec((bt, 1), lambda i: (i, 0)),
            pl.BlockSpec((bt, 1), lambda i: (i, 0)),
        ],
        out_specs=pl.BlockSpec((bt, DIM), lambda i: (i, 0)),
        out_shape=jax.ShapeDtypeStruct((TOKENS, DIM), jnp.float32),
    )(y01, y01, p1.reshape(TOKENS, 1), p2.reshape(TOKENS, 1))


@jax.jit
def _moe(x, gate_w, gate_b, w1, b1, w2, b2):
    e1, e2, p1, p2 = _gate(x, gate_w, gate_b)
    dest2d, bexp = _routing(e1, e2)
    dest3 = dest2d.reshape(NWORKERS, NCH, GD)
    xg = _sc_dispatch(x, dest3)
    y = _ffn(bexp.reshape(NBPAD), xg, w1, b1, w2, b2)
    y01 = _sc_gather_y(y, dest3)
    return _combine_add(y01, p1, p2)


def kernel(x, gate_w, gate_b, w1, b1, w2, b2):
    return _moe(x, gate_w, gate_b, w1, b1, w2, b2)
